# Initial kernel scaffold; baseline (speedup 1.0000x reference)
import jax
import jax.numpy as jnp
from jax.experimental import pallas as pl


def kernel(x, edge_index):
    raise NotImplementedError("write your pallas kernel here")


# speedup vs baseline: 5.6064x; 5.6064x over previous
import functools

import jax
import jax.numpy as jnp
from jax import lax
from jax.experimental import pallas as pl
from jax.experimental.pallas import tpu as pltpu
from jax.experimental.pallas import tpu_sc as plsc

N_NODES = 10000
N_EDGES = 320000
D = 128

NUM_CORES = 2
NUM_SUBCORES = 16
NW = NUM_CORES * NUM_SUBCORES
B_PER_W = N_EDGES // NW
CHUNK = 128
N_FULL = B_PER_W // CHUNK
TAIL = B_PER_W - N_FULL * CHUNK


def _gather_kernel(x_hbm, idx_hbm, out_hbm, idx_v, buf0, buf1,
                   gsem0, gsem1, wsem0, wsem1):
    bufs = (buf0, buf1)
    gsems = (gsem0, gsem1)
    wsems = (wsem0, wsem1)

    wid = lax.axis_index("s") * NUM_CORES + lax.axis_index("c")
    base = wid * B_PER_W

    pltpu.sync_copy(idx_hbm.at[pl.ds(base, B_PER_W)], idx_v)

    def start_gather(g, b):
        pltpu.async_copy(
            x_hbm.at[idx_v.at[pl.ds(g * CHUNK, CHUNK)]], bufs[b], gsems[b])

    def wait_gather(b):
        pltpu.make_async_copy(
            x_hbm.at[idx_v.at[pl.ds(0, CHUNK)]], bufs[b], gsems[b]).wait()

    def start_write(g, b):
        pltpu.async_copy(
            bufs[b], out_hbm.at[pl.ds(base + g * CHUNK, CHUNK)], wsems[b])

    def wait_write(b):
        pltpu.make_async_copy(
            bufs[b], out_hbm.at[pl.ds(base, CHUNK)], wsems[b]).wait()

    pltpu.async_copy(
        x_hbm.at[idx_v.at[pl.ds(N_FULL * CHUNK, TAIL)]],
        buf0.at[pl.ds(0, TAIL)], gsem0)
    pltpu.make_async_copy(
        x_hbm.at[idx_v.at[pl.ds(0, TAIL)]],
        buf0.at[pl.ds(0, TAIL)], gsem0).wait()
    pltpu.sync_copy(buf0.at[pl.ds(0, TAIL)],
                    out_hbm.at[pl.ds(base + N_FULL * CHUNK, TAIL)])

    start_gather(0, 0)
    start_gather(1, 1)

    def body(j, carry):
        for b in range(2):
            g = j * 2 + b
            wait_gather(b)
            start_write(g, b)
            wait_write(b)

            @pl.when(j < N_FULL // 2 - 1)
            def _():
                start_gather(g + 2, b)
        return carry

    lax.fori_loop(0, N_FULL // 2, body, 0)


@jax.jit
def _gather(x, idx):
    mesh = plsc.VectorSubcoreMesh(core_axis_name="c", subcore_axis_name="s")
    return pl.kernel(
        _gather_kernel,
        out_type=jax.ShapeDtypeStruct((N_EDGES, D), jnp.float32),
        mesh=mesh,
        scratch_types=[
            pltpu.VMEM((B_PER_W,), jnp.int32),
            pltpu.VMEM((CHUNK, D), jnp.float32),
            pltpu.VMEM((CHUNK, D), jnp.float32),
            pltpu.SemaphoreType.DMA,
            pltpu.SemaphoreType.DMA,
            pltpu.SemaphoreType.DMA,
            pltpu.SemaphoreType.DMA,
        ],
    )(x, idx)


def kernel(x, edge_index):
    idx = edge_index[0].astype(jnp.int32)
    return _gather(x, idx)


# speedup vs baseline: 5.6914x; 1.0152x over previous
import functools

import jax
import jax.numpy as jnp
from jax import lax
from jax.experimental import pallas as pl
from jax.experimental.pallas import tpu as pltpu
from jax.experimental.pallas import tpu_sc as plsc

N_NODES = 10000
N_EDGES = 320000
D = 128

NUM_CORES = 2
NUM_SUBCORES = 16
NW = NUM_CORES * NUM_SUBCORES
B_PER_W = N_EDGES // NW
SUB = 128
CHUNK = 256
N_FULL = B_PER_W // CHUNK
TAIL = B_PER_W - N_FULL * CHUNK
NBUF = 3


def _gather_kernel(x_hbm, idx_hbm, out_hbm, idx_v, buf0, buf1, buf2,
                   gsem0, gsem1, gsem2, wsem0, wsem1, wsem2):
    bufs = (buf0, buf1, buf2)
    gsems = (gsem0, gsem1, gsem2)
    wsems = (wsem0, wsem1, wsem2)

    wid = lax.axis_index("s") * NUM_CORES + lax.axis_index("c")
    base = wid * B_PER_W

    pltpu.sync_copy(idx_hbm.at[pl.ds(base, B_PER_W)], idx_v)

    def start_gather(g, b):
        pltpu.async_copy(
            x_hbm.at[idx_v.at[pl.ds(g * CHUNK, SUB)]],
            bufs[b].at[pl.ds(0, SUB)], gsems[b])
        pltpu.async_copy(
            x_hbm.at[idx_v.at[pl.ds(g * CHUNK + SUB, SUB)]],
            bufs[b].at[pl.ds(SUB, SUB)], gsems[b])

    def wait_gather(b):
        pltpu.make_async_copy(
            x_hbm.at[idx_v.at[pl.ds(0, CHUNK)]], bufs[b], gsems[b]).wait()

    def start_write(g, b):
        pltpu.async_copy(
            bufs[b], out_hbm.at[pl.ds(base + g * CHUNK, CHUNK)], wsems[b])

    def wait_write(b):
        pltpu.make_async_copy(
            bufs[b], out_hbm.at[pl.ds(base, CHUNK)], wsems[b]).wait()

    pltpu.async_copy(
        x_hbm.at[idx_v.at[pl.ds(N_FULL * CHUNK, TAIL)]],
        buf0.at[pl.ds(0, TAIL)], gsem0)
    pltpu.make_async_copy(
        x_hbm.at[idx_v.at[pl.ds(0, TAIL)]],
        buf0.at[pl.ds(0, TAIL)], gsem0).wait()
    pltpu.sync_copy(buf0.at[pl.ds(0, TAIL)],
                    out_hbm.at[pl.ds(base + N_FULL * CHUNK, TAIL)])

    start_gather(0, 0)

    def body(g, carry):
        for b in range(NBUF):
            is_b_next = (g + 1) % NBUF == b

            @pl.when(jnp.logical_and(is_b_next, g + 1 < N_FULL))
            def _():
                @pl.when(g >= 2)
                def _():
                    wait_write(b)
                start_gather(g + 1, b)

        for b in range(NBUF):
            @pl.when(g % NBUF == b)
            def _():
                wait_gather(b)
                start_write(g, b)
        return carry

    lax.fori_loop(0, N_FULL, body, 0)

    for g in range(N_FULL - NBUF + 1, N_FULL):
        wait_write(g % NBUF)


@jax.jit
def _gather(x, idx):
    mesh = plsc.VectorSubcoreMesh(core_axis_name="c", subcore_axis_name="s")
    return pl.kernel(
        _gather_kernel,
        out_type=jax.ShapeDtypeStruct((N_EDGES, D), jnp.float32),
        mesh=mesh,
        scratch_types=[
            pltpu.VMEM((B_PER_W,), jnp.int32),
            pltpu.VMEM((CHUNK, D), jnp.float32),
            pltpu.VMEM((CHUNK, D), jnp.float32),
            pltpu.VMEM((CHUNK, D), jnp.float32),
            pltpu.SemaphoreType.DMA,
            pltpu.SemaphoreType.DMA,
            pltpu.SemaphoreType.DMA,
            pltpu.SemaphoreType.DMA,
            pltpu.SemaphoreType.DMA,
            pltpu.SemaphoreType.DMA,
        ],
    )(x, idx)


def kernel(x, edge_index):
    idx = edge_index[0].astype(jnp.int32)
    return _gather(x, idx)


# speedup vs baseline: 6.1221x; 1.0757x over previous
import jax
import jax.numpy as jnp
from jax import lax
from jax.experimental import pallas as pl
from jax.experimental.pallas import tpu as pltpu
from jax.experimental.pallas import tpu_sc as plsc

N_NODES = 10000
N_EDGES = 320000
D = 128

NUM_CORES = 2
NUM_SUBCORES = 16
NW = NUM_CORES * NUM_SUBCORES
B_PER_W = N_EDGES // NW
CHUNK = 192
SUB = 96
N_FULL = B_PER_W // CHUNK
TAIL = B_PER_W - N_FULL * CHUNK
NBUF = 4
LOOKAHEAD = 2


def _gather_kernel(x_hbm, idx_hbm, out_hbm, idx_v, buf0, buf1, buf2, buf3,
                   gsem0, gsem1, gsem2, gsem3, wsem0, wsem1, wsem2, wsem3):
    bufs = (buf0, buf1, buf2, buf3)
    gsems = (gsem0, gsem1, gsem2, gsem3)
    wsems = (wsem0, wsem1, wsem2, wsem3)

    wid = lax.axis_index("s") * NUM_CORES + lax.axis_index("c")
    base = wid * B_PER_W

    pltpu.sync_copy(idx_hbm.at[pl.ds(base, B_PER_W)], idx_v)

    def start_gather(g, b):
        pltpu.async_copy(
            x_hbm.at[idx_v.at[pl.ds(g * CHUNK, SUB)]],
            bufs[b].at[pl.ds(0, SUB)], gsems[b])
        pltpu.async_copy(
            x_hbm.at[idx_v.at[pl.ds(g * CHUNK + SUB, SUB)]],
            bufs[b].at[pl.ds(SUB, SUB)], gsems[b])

    def wait_gather(b):
        pltpu.make_async_copy(
            x_hbm.at[idx_v.at[pl.ds(0, CHUNK)]], bufs[b], gsems[b]).wait()

    def start_write(g, b):
        pltpu.async_copy(
            bufs[b], out_hbm.at[pl.ds(base + g * CHUNK, CHUNK)], wsems[b])

    def wait_write(b):
        pltpu.make_async_copy(
            bufs[b], out_hbm.at[pl.ds(base, CHUNK)], wsems[b]).wait()

    pltpu.async_copy(
        x_hbm.at[idx_v.at[pl.ds(N_FULL * CHUNK, TAIL)]],
        buf0.at[pl.ds(0, TAIL)], gsem0)
    pltpu.make_async_copy(
        x_hbm.at[idx_v.at[pl.ds(0, TAIL)]],
        buf0.at[pl.ds(0, TAIL)], gsem0).wait()
    pltpu.sync_copy(buf0.at[pl.ds(0, TAIL)],
                    out_hbm.at[pl.ds(base + N_FULL * CHUNK, TAIL)])

    for g in range(LOOKAHEAD):
        start_gather(g, g)

    def body(j, carry):
        for b in range(NBUF):
            g = j * NBUF + b
            bn = (b + LOOKAHEAD) % NBUF

            @pl.when(g + LOOKAHEAD < N_FULL)
            def _():
                @pl.when(g >= NBUF - LOOKAHEAD)
                def _():
                    wait_write(bn)
                start_gather(g + LOOKAHEAD, bn)

            wait_gather(b)
            start_write(g, b)
        return carry

    lax.fori_loop(0, N_FULL // NBUF, body, 0)

    for g in range(N_FULL - NBUF, N_FULL):
        wait_write(g % NBUF)


@jax.jit
def _gather(x, idx):
    mesh = plsc.VectorSubcoreMesh(core_axis_name="c", subcore_axis_name="s")
    return pl.kernel(
        _gather_kernel,
        out_type=jax.ShapeDtypeStruct((N_EDGES, D), jnp.float32),
        mesh=mesh,
        scratch_types=[
            pltpu.VMEM((B_PER_W,), jnp.int32),
            pltpu.VMEM((CHUNK, D), jnp.float32),
            pltpu.VMEM((CHUNK, D), jnp.float32),
            pltpu.VMEM((CHUNK, D), jnp.float32),
            pltpu.VMEM((CHUNK, D), jnp.float32),
            pltpu.SemaphoreType.DMA,
            pltpu.SemaphoreType.DMA,
            pltpu.SemaphoreType.DMA,
            pltpu.SemaphoreType.DMA,
            pltpu.SemaphoreType.DMA,
            pltpu.SemaphoreType.DMA,
            pltpu.SemaphoreType.DMA,
            pltpu.SemaphoreType.DMA,
        ],
    )(x, idx)


def kernel(x, edge_index):
    return _gather(x, edge_index.astype(jnp.int32).reshape(-1))


# speedup vs baseline: 6.1502x; 1.0046x over previous
import jax
import jax.numpy as jnp
from jax import lax
from jax.experimental import pallas as pl
from jax.experimental.pallas import tpu as pltpu
from jax.experimental.pallas import tpu_sc as plsc

N_NODES = 10000
N_EDGES = 320000
D = 128

NUM_CORES = 2
NUM_SUBCORES = 16
NW = NUM_CORES * NUM_SUBCORES
B_PER_W = N_EDGES // NW
CHUNK = 200
SUBS = ((0, 200),)
N_FULL = B_PER_W // CHUNK
TAIL = B_PER_W - N_FULL * CHUNK
NBUF = 2
LOOKAHEAD = 1


def _gather_kernel(x_hbm, idx_hbm, out_hbm, idx_v, buf0, buf1,
                   gsem0, gsem1, wsem0, wsem1):
    bufs = (buf0, buf1)
    gsems = (gsem0, gsem1)
    wsems = (wsem0, wsem1)

    wid = lax.axis_index("s") * NUM_CORES + lax.axis_index("c")
    base = wid * B_PER_W

    pltpu.sync_copy(idx_hbm.at[pl.ds(base, B_PER_W)], idx_v)

    def start_gather(g, b):
        for off, n in SUBS:
            pltpu.async_copy(
                x_hbm.at[idx_v.at[pl.ds(g * CHUNK + off, n)]],
                bufs[b].at[pl.ds(off, n)], gsems[b])

    def wait_gather(b):
        pltpu.make_async_copy(
            x_hbm.at[idx_v.at[pl.ds(0, CHUNK)]], bufs[b], gsems[b]).wait()

    def start_write(g, b):
        pltpu.async_copy(
            bufs[b], out_hbm.at[pl.ds(base + g * CHUNK, CHUNK)], wsems[b])

    def wait_write(b):
        pltpu.make_async_copy(
            bufs[b], out_hbm.at[pl.ds(base, CHUNK)], wsems[b]).wait()


    for g in range(LOOKAHEAD):
        start_gather(g, g)

    def body(j, carry):
        for b in range(NBUF):
            g = j * NBUF + b
            bn = (b + LOOKAHEAD) % NBUF

            @pl.when(g + LOOKAHEAD < N_FULL)
            def _():
                @pl.when(g >= NBUF - LOOKAHEAD)
                def _():
                    wait_write(bn)
                start_gather(g + LOOKAHEAD, bn)

            wait_gather(b)
            start_write(g, b)
        return carry

    lax.fori_loop(0, N_FULL // NBUF, body, 0)

    for g in range(N_FULL - NBUF, N_FULL):
        wait_write(g % NBUF)


@jax.jit
def _gather(x, idx):
    mesh = plsc.VectorSubcoreMesh(core_axis_name="c", subcore_axis_name="s")
    return pl.kernel(
        _gather_kernel,
        out_type=jax.ShapeDtypeStruct((N_EDGES, D), jnp.float32),
        mesh=mesh,
        scratch_types=[
            pltpu.VMEM((B_PER_W,), jnp.int32),
            pltpu.VMEM((CHUNK, D), jnp.float32),
            pltpu.VMEM((CHUNK, D), jnp.float32),
            pltpu.SemaphoreType.DMA,
            pltpu.SemaphoreType.DMA,
            pltpu.SemaphoreType.DMA,
            pltpu.SemaphoreType.DMA,
        ],
    )(x, idx)


def kernel(x, edge_index):
    return _gather(x, edge_index.astype(jnp.int32).reshape(-1))
